# baseline (speedup 1.0000x reference)
import functools

import jax
import jax.numpy as jnp
from jax import lax
from jax.experimental import pallas as pl
from jax.experimental.pallas import tpu as pltpu
from jax.experimental.pallas import tpu_sc as plsc

_NC = 2
_NS = 16
_NW = _NC * _NS

_IVEC = 128
_TB = 2048


@functools.lru_cache(maxsize=None)
def _make_repack(v: int, d: int):
    assert d == 64
    grid = (v + _TB - 1) // _TB
    out_rows = grid * _TB // 2

    def body(in_ref, out_ref):
        ident = jnp.eye(d, dtype=jnp.float32)
        dn = (((0,), (0,)), ((), ()))
        a = lax.dot_general(in_ref[:, : _TB // 2], ident, dn,
                            preferred_element_type=jnp.float32)
        c = lax.dot_general(in_ref[:, _TB // 2 :], ident, dn,
                            preferred_element_type=jnp.float32)
        out_ref[...] = jnp.concatenate([a, c], axis=1)

    return pl.pallas_call(
        body,
        grid=(grid,),
        in_specs=[pl.BlockSpec((d, _TB), lambda i: (0, i))],
        out_specs=pl.BlockSpec((_TB // 2, 2 * d), lambda i: (i, 0)),
        out_shape=jax.ShapeDtypeStruct((out_rows, 2 * d), jnp.float32),
    )


@functools.lru_cache(maxsize=None)
def _make_gather(n: int, vpad: int, d: int, chunk: int):
    per_w = n // _NW
    n_chunks = per_w // chunk
    u = chunk // _IVEC
    assert per_w % chunk == 0 and chunk % _IVEC == 0

    mesh = plsc.VectorSubcoreMesh(core_axis_name="c", subcore_axis_name="s")

    @functools.partial(
        pl.kernel,
        out_type=jax.ShapeDtypeStruct((n, 2 * d), jnp.float32),
        mesh=mesh,
        compiler_params=pltpu.CompilerParams(use_tc_tiling_on_sc=False),
        scratch_types=[
            pltpu.VMEM((4, chunk), jnp.int32),
            pltpu.VMEM((3, chunk, d), jnp.float32),
            pltpu.SemaphoreType.DMA((4,)),
            pltpu.SemaphoreType.DMA((3,)),
            pltpu.SemaphoreType.DMA((3,)),
        ],
    )
    def gather_kernel(idx_hbm, table_hbm, out_hbm, idx_v, rows_v,
                      idx_sem, g_sem, o_sem):
        wid = lax.axis_index("s") * _NC + lax.axis_index("c")
        base = wid * per_w

        def start_idx_copy(g, s):
            pltpu.async_copy(
                idx_hbm.at[pl.ds(base + g * chunk, chunk)],
                idx_v.at[s],
                idx_sem.at[s],
            )

        def fire_gathers(g, s, b):
            for j in range(u):
                pltpu.async_copy(
                    table_hbm.at[idx_v.at[s, pl.ds(j * _IVEC, _IVEC)]],
                    rows_v.at[b, pl.ds(j * _IVEC, _IVEC)],
                    g_sem.at[b],
                )

        def drain_gathers(b):
            pltpu.make_async_copy(
                rows_v.at[b], out_hbm.at[pl.ds(0, chunk), pl.ds(0, d)],
                g_sem.at[b],
            ).wait()

        def start_out_copy(g, b):
            pltpu.async_copy(
                rows_v.at[b],
                out_hbm.at[pl.ds(base + g * chunk, chunk), pl.ds(0, d)],
                o_sem.at[b],
            )

        def wait_out_copy(b):
            pltpu.make_async_copy(
                rows_v.at[b], out_hbm.at[pl.ds(0, chunk), pl.ds(0, d)],
                o_sem.at[b],
            ).wait()

        for g in range(4):
            start_idx_copy(g, g)

        def body(g, _):
            s = lax.rem(g, 4)
            b = lax.rem(g, 3)
            pltpu.make_async_copy(
                idx_hbm.at[pl.ds(0, chunk)], idx_v.at[s], idx_sem.at[s]
            ).wait()
            @pl.when(g >= 3)
            def _():
                wait_out_copy(b)
            fire_gathers(g, s, b)
            @pl.when(g >= 1)
            def _():
                bp = lax.rem(g + 2, 3)
                sp = lax.rem(g + 3, 4)
                drain_gathers(bp)
                @pl.when(g + 3 < n_chunks)
                def _():
                    start_idx_copy(g + 3, sp)
                start_out_copy(g - 1, bp)
            return _

        lax.fori_loop(0, n_chunks, body, None, unroll=False)

        bl = lax.rem(n_chunks - 1, 3)
        drain_gathers(bl)
        start_out_copy(n_chunks - 1, bl)
        for b in range(3):
            wait_out_copy(b)

    return gather_kernel


def kernel(word_input, weight_all):
    b, l = word_input.shape
    v, d = weight_all.shape
    n = b * l
    idx = word_input.reshape(n)
    h = _TB // 2
    j = idx % _TB
    idx_r = 2 * ((idx // _TB) * h + j % h) + j // h
    tbl2 = _make_repack(v, d)(weight_all.T)
    vpad = tbl2.shape[0] * 2
    tbl = tbl2.reshape(vpad, d)
    out2 = _make_gather(n, vpad, d, 640)(idx_r, tbl)
    return out2[:, :d].reshape(b, l, d)


# speedup vs baseline: 1.3293x; 1.3293x over previous
import functools

import jax
import jax.numpy as jnp
from jax import lax
from jax.experimental import pallas as pl
from jax.experimental.pallas import tpu as pltpu
from jax.experimental.pallas import tpu_sc as plsc

_NC = 2
_NS = 16
_NW = _NC * _NS

_IVEC = 128
_TB = 8192


@functools.lru_cache(maxsize=None)
def _make_repack(v: int, d: int):
    assert d == 64
    grid = (v + _TB - 1) // _TB
    out_rows = grid * _TB // 2

    def body(in_ref, out_ref):
        a = jnp.transpose(in_ref[:, : _TB // 2], (1, 0))
        c = jnp.transpose(in_ref[:, _TB // 2 :], (1, 0))
        out_ref[...] = jnp.concatenate([a, c], axis=1)

    return pl.pallas_call(
        body,
        grid=(grid,),
        in_specs=[pl.BlockSpec((d, _TB), lambda i: (0, i))],
        out_specs=pl.BlockSpec((_TB // 2, 2 * d), lambda i: (i, 0)),
        out_shape=jax.ShapeDtypeStruct((out_rows, 2 * d), jnp.float32),
    )


@functools.lru_cache(maxsize=None)
def _make_gather(n: int, vpad: int, d: int, chunk: int):
    per_w = n // _NW
    n_chunks = per_w // chunk
    u = chunk // _IVEC
    assert per_w % chunk == 0 and chunk % _IVEC == 0

    mesh = plsc.VectorSubcoreMesh(core_axis_name="c", subcore_axis_name="s")

    @functools.partial(
        pl.kernel,
        out_type=jax.ShapeDtypeStruct((n, 2 * d), jnp.float32),
        mesh=mesh,
        compiler_params=pltpu.CompilerParams(use_tc_tiling_on_sc=False),
        scratch_types=[
            pltpu.VMEM((4, chunk), jnp.int32),
            pltpu.VMEM((3, chunk, d), jnp.float32),
            pltpu.SemaphoreType.DMA((4,)),
            pltpu.SemaphoreType.DMA((3,)),
            pltpu.SemaphoreType.DMA((3,)),
        ],
    )
    def gather_kernel(idx_hbm, table_hbm, out_hbm, idx_v, rows_v,
                      idx_sem, g_sem, o_sem):
        wid = lax.axis_index("s") * _NC + lax.axis_index("c")
        base = wid * per_w

        def start_idx_copy(g, s):
            pltpu.async_copy(
                idx_hbm.at[pl.ds(base + g * chunk, chunk)],
                idx_v.at[s],
                idx_sem.at[s],
            )

        def fire_gathers(g, s, b):
            for j in range(u):
                pltpu.async_copy(
                    table_hbm.at[idx_v.at[s, pl.ds(j * _IVEC, _IVEC)]],
                    rows_v.at[b, pl.ds(j * _IVEC, _IVEC)],
                    g_sem.at[b],
                )

        def drain_gathers(b):
            pltpu.make_async_copy(
                rows_v.at[b], out_hbm.at[pl.ds(0, chunk), pl.ds(0, d)],
                g_sem.at[b],
            ).wait()

        def start_out_copy(g, b):
            pltpu.async_copy(
                rows_v.at[b],
                out_hbm.at[pl.ds(base + g * chunk, chunk), pl.ds(0, d)],
                o_sem.at[b],
            )

        def wait_out_copy(b):
            pltpu.make_async_copy(
                rows_v.at[b], out_hbm.at[pl.ds(0, chunk), pl.ds(0, d)],
                o_sem.at[b],
            ).wait()

        for g in range(4):
            start_idx_copy(g, g)

        def body(g, _):
            s = lax.rem(g, 4)
            b = lax.rem(g, 3)
            pltpu.make_async_copy(
                idx_hbm.at[pl.ds(0, chunk)], idx_v.at[s], idx_sem.at[s]
            ).wait()
            @pl.when(g >= 3)
            def _():
                wait_out_copy(b)
            fire_gathers(g, s, b)
            @pl.when(g >= 1)
            def _():
                bp = lax.rem(g + 2, 3)
                sp = lax.rem(g + 3, 4)
                drain_gathers(bp)
                @pl.when(g + 3 < n_chunks)
                def _():
                    start_idx_copy(g + 3, sp)
                start_out_copy(g - 1, bp)
            return _

        lax.fori_loop(0, n_chunks, body, None, unroll=False)

        bl = lax.rem(n_chunks - 1, 3)
        drain_gathers(bl)
        start_out_copy(n_chunks - 1, bl)
        for b in range(3):
            wait_out_copy(b)

    return gather_kernel


def kernel(word_input, weight_all):
    b, l = word_input.shape
    v, d = weight_all.shape
    n = b * l
    idx = word_input.reshape(n)
    h = _TB // 2
    j = idx % _TB
    idx_r = 2 * ((idx // _TB) * h + j % h) + j // h
    tbl2 = _make_repack(v, d)(weight_all.T)
    vpad = tbl2.shape[0] * 2
    tbl = tbl2.reshape(vpad, d)
    out2 = _make_gather(n, vpad, d, 640)(idx_r, tbl)
    return out2[:, :d].reshape(b, l, d)


# speedup vs baseline: 1.4052x; 1.0571x over previous
import functools

import jax
import jax.numpy as jnp
from jax import lax
from jax.experimental import pallas as pl
from jax.experimental.pallas import tpu as pltpu
from jax.experimental.pallas import tpu_sc as plsc

_NC = 2
_NS = 16
_NW = _NC * _NS

_IVEC = 128
_TB = 16384


@functools.lru_cache(maxsize=None)
def _make_repack(v: int, d: int):
    assert d == 64
    grid = (v + _TB - 1) // _TB
    out_rows = grid * _TB // 2

    def body(in_ref, out_ref):
        a = jnp.transpose(in_ref[:, : _TB // 2], (1, 0))
        c = jnp.transpose(in_ref[:, _TB // 2 :], (1, 0))
        out_ref[...] = jnp.concatenate([a, c], axis=1)

    return pl.pallas_call(
        body,
        grid=(grid,),
        in_specs=[pl.BlockSpec((d, _TB), lambda i: (0, i))],
        out_specs=pl.BlockSpec((_TB // 2, 2 * d), lambda i: (i, 0)),
        out_shape=jax.ShapeDtypeStruct((out_rows, 2 * d), jnp.float32),
    )


@functools.lru_cache(maxsize=None)
def _make_gather(n: int, vpad: int, d: int, chunk: int):
    per_w = n // _NW
    n_chunks = per_w // chunk
    u = chunk // _IVEC
    assert per_w % chunk == 0 and chunk % _IVEC == 0

    mesh = plsc.VectorSubcoreMesh(core_axis_name="c", subcore_axis_name="s")

    @functools.partial(
        pl.kernel,
        out_type=jax.ShapeDtypeStruct((n, 2 * d), jnp.float32),
        mesh=mesh,
        compiler_params=pltpu.CompilerParams(use_tc_tiling_on_sc=False),
        scratch_types=[
            pltpu.VMEM((4, chunk), jnp.int32),
            pltpu.VMEM((3, chunk, d), jnp.float32),
            pltpu.SemaphoreType.DMA((4,)),
            pltpu.SemaphoreType.DMA((3,)),
            pltpu.SemaphoreType.DMA((3,)),
        ],
    )
    def gather_kernel(idx_hbm, table_hbm, out_hbm, idx_v, rows_v,
                      idx_sem, g_sem, o_sem):
        wid = lax.axis_index("s") * _NC + lax.axis_index("c")
        base = wid * per_w

        def start_idx_copy(g, s):
            pltpu.async_copy(
                idx_hbm.at[pl.ds(base + g * chunk, chunk)],
                idx_v.at[s],
                idx_sem.at[s],
            )

        def fire_gathers(g, s, b):
            for j in range(u):
                pltpu.async_copy(
                    table_hbm.at[idx_v.at[s, pl.ds(j * _IVEC, _IVEC)]],
                    rows_v.at[b, pl.ds(j * _IVEC, _IVEC)],
                    g_sem.at[b],
                )

        def drain_gathers(b):
            pltpu.make_async_copy(
                rows_v.at[b], out_hbm.at[pl.ds(0, chunk), pl.ds(0, d)],
                g_sem.at[b],
            ).wait()

        def start_out_copy(g, b):
            pltpu.async_copy(
                rows_v.at[b],
                out_hbm.at[pl.ds(base + g * chunk, chunk), pl.ds(0, d)],
                o_sem.at[b],
            )

        def wait_out_copy(b):
            pltpu.make_async_copy(
                rows_v.at[b], out_hbm.at[pl.ds(0, chunk), pl.ds(0, d)],
                o_sem.at[b],
            ).wait()

        for g in range(4):
            start_idx_copy(g, g)

        def body(g, _):
            s = lax.rem(g, 4)
            b = lax.rem(g, 3)
            pltpu.make_async_copy(
                idx_hbm.at[pl.ds(0, chunk)], idx_v.at[s], idx_sem.at[s]
            ).wait()
            @pl.when(g >= 3)
            def _():
                wait_out_copy(b)
            fire_gathers(g, s, b)
            @pl.when(g >= 1)
            def _():
                bp = lax.rem(g + 2, 3)
                sp = lax.rem(g + 3, 4)
                drain_gathers(bp)
                @pl.when(g + 3 < n_chunks)
                def _():
                    start_idx_copy(g + 3, sp)
                start_out_copy(g - 1, bp)
            return _

        lax.fori_loop(0, n_chunks, body, None, unroll=False)

        bl = lax.rem(n_chunks - 1, 3)
        drain_gathers(bl)
        start_out_copy(n_chunks - 1, bl)
        for b in range(3):
            wait_out_copy(b)

    return gather_kernel


def kernel(word_input, weight_all):
    b, l = word_input.shape
    v, d = weight_all.shape
    n = b * l
    idx = word_input.reshape(n)
    h = _TB // 2
    j = idx % _TB
    idx_r = 2 * ((idx // _TB) * h + j % h) + j // h
    tbl2 = _make_repack(v, d)(weight_all.T)
    vpad = tbl2.shape[0] * 2
    tbl = tbl2.reshape(vpad, d)
    out2 = _make_gather(n, vpad, d, 640)(idx_r, tbl)
    return out2[:, :d].reshape(b, l, d)


# speedup vs baseline: 1.4437x; 1.0274x over previous
import functools

import jax
import jax.numpy as jnp
from jax import lax
from jax.experimental import pallas as pl
from jax.experimental.pallas import tpu as pltpu
from jax.experimental.pallas import tpu_sc as plsc

_NC = 2
_NS = 16
_NW = _NC * _NS

_IVEC = 128
_TB = 32768


@functools.lru_cache(maxsize=None)
def _make_repack(v: int, d: int):
    assert d == 64
    grid = (v + _TB - 1) // _TB
    out_rows = grid * _TB // 2

    def body(in_ref, out_ref):
        a = jnp.transpose(in_ref[:, : _TB // 2], (1, 0))
        c = jnp.transpose(in_ref[:, _TB // 2 :], (1, 0))
        out_ref[...] = jnp.concatenate([a, c], axis=1)

    return pl.pallas_call(
        body,
        grid=(grid,),
        in_specs=[pl.BlockSpec((d, _TB), lambda i: (0, i))],
        out_specs=pl.BlockSpec((_TB // 2, 2 * d), lambda i: (i, 0)),
        out_shape=jax.ShapeDtypeStruct((out_rows, 2 * d), jnp.float32),
    )


@functools.lru_cache(maxsize=None)
def _make_gather(n: int, vpad: int, d: int, chunk: int):
    per_w = n // _NW
    n_chunks = per_w // chunk
    u = chunk // _IVEC
    assert per_w % chunk == 0 and chunk % _IVEC == 0

    mesh = plsc.VectorSubcoreMesh(core_axis_name="c", subcore_axis_name="s")

    @functools.partial(
        pl.kernel,
        out_type=jax.ShapeDtypeStruct((n, 2 * d), jnp.float32),
        mesh=mesh,
        compiler_params=pltpu.CompilerParams(use_tc_tiling_on_sc=False),
        scratch_types=[
            pltpu.VMEM((4, chunk), jnp.int32),
            pltpu.VMEM((3, chunk, d), jnp.float32),
            pltpu.SemaphoreType.DMA((4,)),
            pltpu.SemaphoreType.DMA((3,)),
            pltpu.SemaphoreType.DMA((3,)),
        ],
    )
    def gather_kernel(idx_hbm, table_hbm, out_hbm, idx_v, rows_v,
                      idx_sem, g_sem, o_sem):
        wid = lax.axis_index("s") * _NC + lax.axis_index("c")
        base = wid * per_w

        def start_idx_copy(g, s):
            pltpu.async_copy(
                idx_hbm.at[pl.ds(base + g * chunk, chunk)],
                idx_v.at[s],
                idx_sem.at[s],
            )

        def fire_gathers(g, s, b):
            for j in range(u):
                pltpu.async_copy(
                    table_hbm.at[idx_v.at[s, pl.ds(j * _IVEC, _IVEC)]],
                    rows_v.at[b, pl.ds(j * _IVEC, _IVEC)],
                    g_sem.at[b],
                )

        def drain_gathers(b):
            pltpu.make_async_copy(
                rows_v.at[b], out_hbm.at[pl.ds(0, chunk), pl.ds(0, d)],
                g_sem.at[b],
            ).wait()

        def start_out_copy(g, b):
            pltpu.async_copy(
                rows_v.at[b],
                out_hbm.at[pl.ds(base + g * chunk, chunk), pl.ds(0, d)],
                o_sem.at[b],
            )

        def wait_out_copy(b):
            pltpu.make_async_copy(
                rows_v.at[b], out_hbm.at[pl.ds(0, chunk), pl.ds(0, d)],
                o_sem.at[b],
            ).wait()

        for g in range(4):
            start_idx_copy(g, g)

        def body(g, _):
            s = lax.rem(g, 4)
            b = lax.rem(g, 3)
            pltpu.make_async_copy(
                idx_hbm.at[pl.ds(0, chunk)], idx_v.at[s], idx_sem.at[s]
            ).wait()
            @pl.when(g >= 3)
            def _():
                wait_out_copy(b)
            fire_gathers(g, s, b)
            @pl.when(g >= 1)
            def _():
                bp = lax.rem(g + 2, 3)
                sp = lax.rem(g + 3, 4)
                drain_gathers(bp)
                @pl.when(g + 3 < n_chunks)
                def _():
                    start_idx_copy(g + 3, sp)
                start_out_copy(g - 1, bp)
            return _

        lax.fori_loop(0, n_chunks, body, None, unroll=False)

        bl = lax.rem(n_chunks - 1, 3)
        drain_gathers(bl)
        start_out_copy(n_chunks - 1, bl)
        for b in range(3):
            wait_out_copy(b)

    return gather_kernel


def kernel(word_input, weight_all):
    b, l = word_input.shape
    v, d = weight_all.shape
    n = b * l
    idx = word_input.reshape(n)
    h = _TB // 2
    j = idx % _TB
    idx_r = 2 * ((idx // _TB) * h + j % h) + j // h
    tbl2 = _make_repack(v, d)(weight_all.T)
    vpad = tbl2.shape[0] * 2
    tbl = tbl2.reshape(vpad, d)
    out2 = _make_gather(n, vpad, d, 640)(idx_r, tbl)
    return out2[:, :d].reshape(b, l, d)
